# baseline (speedup 1.0000x reference)
import functools

import jax
import jax.numpy as jnp
from jax import lax
from jax.experimental import pallas as pl
from jax.experimental.pallas import tpu as pltpu
from jax.experimental.pallas import tpu_sc as plsc

BATCH = 16384
EMBED_DIM = 32
NUM_CORES = 2
NUM_SUBCORES = 16
NUM_WORKERS = NUM_CORES * NUM_SUBCORES
B_PER_W = BATCH // NUM_WORKERS
LANES = 16


def _dense_body(stats_ref, w_ref, b_ref, out_ref):
    out_ref[...] = lax.dot_general(
        stats_ref[...], w_ref[...],
        dimension_numbers=(((1,), (1,)), ((), ())),
        preferred_element_type=jnp.float32,
    ) + b_ref[...]


def _dense(team_stats, W, b):
    return pl.pallas_call(
        _dense_body,
        out_shape=jax.ShapeDtypeStruct((BATCH, EMBED_DIM), jnp.float32),
    )(team_stats, W, b.reshape(1, EMBED_DIM))


def _sc_gather_add(emb_table, team_ids, stats_emb):
    mesh = plsc.VectorSubcoreMesh(core_axis_name="c", subcore_axis_name="s")

    @functools.partial(
        pl.kernel,
        mesh=mesh,
        out_type=jax.ShapeDtypeStruct((BATCH, EMBED_DIM), jnp.float32),
        scratch_types=[
            pltpu.VMEM((B_PER_W,), jnp.int32),
            pltpu.VMEM((B_PER_W, EMBED_DIM), jnp.float32),
            pltpu.VMEM((B_PER_W, EMBED_DIM), jnp.float32),
            pltpu.SemaphoreType.DMA,
        ],
        compiler_params=pltpu.CompilerParams(use_tc_tiling_on_sc=False),
    )
    def k(table_hbm, idx_hbm, se_hbm, out_hbm, idx_v, rows_v, se_v, sem):
        wid = lax.axis_index("s") * NUM_CORES + lax.axis_index("c")
        base = wid * B_PER_W
        pltpu.sync_copy(idx_hbm.at[pl.ds(base, B_PER_W)], idx_v)
        gat = pltpu.async_copy(table_hbm.at[idx_v], rows_v, sem)
        pltpu.sync_copy(se_hbm.at[pl.ds(base, B_PER_W)], se_v)
        gat.wait()

        def add_row(r, carry):
            for h in range(EMBED_DIM // LANES):
                sl = pl.ds(h * LANES, LANES)
                rows_v[r, sl] = rows_v[r, sl] + se_v[r, sl]
            return carry

        lax.fori_loop(0, B_PER_W, add_row, 0)
        pltpu.sync_copy(rows_v, out_hbm.at[pl.ds(base, B_PER_W)])

    return k(emb_table, team_ids, stats_emb)


def kernel(team_ids, team_stats, emb_table, W, b):
    stats_emb = _dense(team_stats, W, b)
    return _sc_gather_add(emb_table, team_ids.astype(jnp.int32), stats_emb)
